# Initial kernel scaffold; baseline (speedup 1.0000x reference)
import jax
import jax.numpy as jnp
from jax.experimental import pallas as pl


def kernel(x, edge_index, Wp, bp, W1, W2, Wc, bc):
    raise NotImplementedError("write your pallas kernel here")


# speedup vs baseline: 10.5725x; 10.5725x over previous
import functools

import numpy as np
import jax
import jax.numpy as jnp
from jax import lax
from jax.experimental import pallas as pl
from jax.experimental.pallas import tpu as pltpu
from jax.experimental.pallas import tpu_sc as plsc

_N = 10000
_D = 128
_O = 64
_NPAD = 10240
_E = 320000
_CH = 128
_TILES = 32
_EPT = ((_E + _TILES * _CH - 1) // (_TILES * _CH)) * _CH
_EPAD = _EPT * _TILES
_NCH = _EPT // _CH
_RPT = _NPAD // 16

_ALPHA = 0.1
_B1 = np.float32(np.log(0.5 / 1 + 1.0))
_B2 = np.float32(np.log(0.5 / 2 + 1.0))

_R1 = 1280
_R2 = 2000


def _sc_degree(dst_p):
    mesh = plsc.VectorSubcoreMesh(core_axis_name="c", subcore_axis_name="s")

    @functools.partial(
        pl.kernel,
        mesh=mesh,
        out_type=jax.ShapeDtypeStruct((2 * _NPAD,), jnp.float32),
        scratch_types=[
            pltpu.VMEM_SHARED((_NPAD,), jnp.float32),
            pltpu.VMEM((_CH,), jnp.int32),
            pltpu.VMEM((_CH,), jnp.float32),
            pltpu.VMEM((_RPT,), jnp.float32),
        ],
    )
    def k(dst_hbm, out_hbm, deg, dstv, ones, zbuf):
        c = lax.axis_index("c")
        s = lax.axis_index("s")
        wid = c * 16 + s
        one16 = jnp.full((16,), 1.0, jnp.float32)
        zero16 = jnp.zeros((16,), jnp.float32)
        for j in range(_CH // 16):
            ones[pl.ds(j * 16, 16)] = one16

        def zfill(i, _):
            zbuf[pl.ds(i * 16, 16)] = zero16
            return 0

        lax.fori_loop(0, _RPT // 16, zfill, 0)
        row0 = s * _RPT
        pltpu.sync_copy(zbuf, deg.at[pl.ds(row0, _RPT)])
        plsc.subcore_barrier()

        ebase = wid * _EPT

        def body(j, _):
            b = ebase + j * _CH
            pltpu.sync_copy(dst_hbm.at[pl.ds(b, _CH)], dstv)
            pltpu.sync_copy(ones, deg.at[dstv], add=True)
            return 0

        lax.fori_loop(0, _NCH, body, 0)
        plsc.subcore_barrier()
        pltpu.sync_copy(deg.at[pl.ds(row0, _RPT)],
                        out_hbm.at[pl.ds(c * _NPAD + row0, _RPT)])

    return k(dst_p)


def _sc_propagate(xs, src_p, dst_p):
    mesh = plsc.VectorSubcoreMesh(core_axis_name="c", subcore_axis_name="s")

    @functools.partial(
        pl.kernel,
        mesh=mesh,
        out_type=jax.ShapeDtypeStruct((2 * _NPAD, _D), jnp.float32),
        scratch_types=[
            pltpu.VMEM_SHARED((_NPAD, _D), jnp.float32),
            pltpu.VMEM((_CH,), jnp.int32),
            pltpu.VMEM((_CH,), jnp.int32),
            pltpu.VMEM((_CH, _D), jnp.float32),
            pltpu.SemaphoreType.DMA,
        ],
    )
    def k(xs_hbm, src_hbm, dst_hbm, out_hbm, acc, srcv, dstv, rows, sem):
        c = lax.axis_index("c")
        s = lax.axis_index("s")
        wid = c * 16 + s
        zero16 = jnp.zeros((16,), jnp.float32)

        def zrow(i, _):
            for j in range(_D // 16):
                rows[i, pl.ds(j * 16, 16)] = zero16
            return 0

        lax.fori_loop(0, _CH, zrow, 0)
        row0 = s * _RPT
        for kblk in range(_RPT // _CH):
            pltpu.sync_copy(rows, acc.at[pl.ds(row0 + kblk * _CH, _CH)])
        plsc.subcore_barrier()

        ebase = wid * _EPT

        def body(j, _):
            b = ebase + j * _CH
            pltpu.sync_copy(src_hbm.at[pl.ds(b, _CH)], srcv)
            pltpu.sync_copy(dst_hbm.at[pl.ds(b, _CH)], dstv)
            pltpu.async_copy(xs_hbm.at[srcv], rows, sem).wait()
            pltpu.sync_copy(rows, acc.at[dstv], add=True)
            return 0

        lax.fori_loop(0, _NCH, body, 0)
        plsc.subcore_barrier()
        pltpu.sync_copy(acc.at[pl.ds(row0, _RPT)],
                        out_hbm.at[pl.ds(c * _NPAD + row0, _RPT)])

    return k(xs, src_p, dst_p)


def _dense0(x_pad, Wp, bp2, deg3):
    def body(x_ref, wp_ref, bp_ref, deg_ref, x0_ref, xs0_ref, dinv_ref):
        pid = pl.program_id(0)
        x0 = jnp.maximum(jnp.dot(x_ref[...], wp_ref[...]) + bp_ref[...], 0.0)
        deg = deg_ref[0] + deg_ref[1]
        rid = lax.broadcasted_iota(jnp.int32, (_R1, 1), 0) + pid * _R1
        deg = deg + jnp.where(rid < _N, 1.0, 0.0)
        dinv = jnp.where(deg > 0, lax.rsqrt(deg), 0.0)
        x0_ref[...] = x0
        xs0_ref[...] = x0 * dinv
        dinv_ref[...] = dinv

    grid = _NPAD // _R1
    return pl.pallas_call(
        body,
        grid=(grid,),
        in_specs=[
            pl.BlockSpec((_R1, _D), lambda r: (r, 0)),
            pl.BlockSpec((_D, _D), lambda r: (0, 0)),
            pl.BlockSpec((1, _D), lambda r: (0, 0)),
            pl.BlockSpec((2, _R1, 1), lambda r: (0, r, 0)),
        ],
        out_specs=[
            pl.BlockSpec((_R1, _D), lambda r: (r, 0)),
            pl.BlockSpec((_R1, _D), lambda r: (r, 0)),
            pl.BlockSpec((_R1, 1), lambda r: (r, 0)),
        ],
        out_shape=[
            jax.ShapeDtypeStruct((_NPAD, _D), jnp.float32),
            jax.ShapeDtypeStruct((_NPAD, _D), jnp.float32),
            jax.ShapeDtypeStruct((_NPAD, 1), jnp.float32),
        ],
    )(x_pad, Wp, bp2, deg3)


def _combine1(acc3, xs0, x0, dinv, W1):
    def body(acc_ref, xs0_ref, x0_ref, dinv_ref, w1_ref, xs1_ref):
        dv = dinv_ref[...]
        prop = (acc_ref[0] + acc_ref[1] + xs0_ref[...]) * dv
        h = (1.0 - _ALPHA) * prop + _ALPHA * x0_ref[...]
        h = (1.0 - _B1) * h + _B1 * jnp.dot(h, w1_ref[...])
        xs1_ref[...] = jnp.maximum(h, 0.0) * dv

    grid = _NPAD // _R1
    return pl.pallas_call(
        body,
        grid=(grid,),
        in_specs=[
            pl.BlockSpec((2, _R1, _D), lambda r: (0, r, 0)),
            pl.BlockSpec((_R1, _D), lambda r: (r, 0)),
            pl.BlockSpec((_R1, _D), lambda r: (r, 0)),
            pl.BlockSpec((_R1, 1), lambda r: (r, 0)),
            pl.BlockSpec((_D, _D), lambda r: (0, 0)),
        ],
        out_specs=pl.BlockSpec((_R1, _D), lambda r: (r, 0)),
        out_shape=jax.ShapeDtypeStruct((_NPAD, _D), jnp.float32),
    )(acc3, xs0, x0, dinv, W1)


def _head(acc3, xs1, x0, dinv, W2, Wc, bc2):
    def body(acc_ref, xs1_ref, x0_ref, dinv_ref, w2_ref, wc_ref, bc_ref,
             lg_ref, emb_ref, sm_ref, hd_ref):
        dv = dinv_ref[...]
        prop = (acc_ref[0] + acc_ref[1] + xs1_ref[...]) * dv
        h = (1.0 - _ALPHA) * prop + _ALPHA * x0_ref[...]
        h = (1.0 - _B2) * h + _B2 * jnp.dot(h, w2_ref[...])
        emb = jnp.maximum(h, 0.0)
        logits = jnp.dot(emb, wc_ref[...]) + bc_ref[...]
        m = jnp.max(logits, axis=1, keepdims=True)
        e = jnp.exp(logits - m)
        sm = e / jnp.sum(e, axis=1, keepdims=True)
        ii = lax.broadcasted_iota(jnp.int32, (_R2, _O), 1)
        hd = jnp.min(jnp.where(logits == m, ii, _O), axis=1, keepdims=True)
        lg_ref[...] = logits
        emb_ref[...] = emb
        sm_ref[...] = sm
        hd_ref[...] = hd

    grid = _N // _R2
    return pl.pallas_call(
        body,
        grid=(grid,),
        in_specs=[
            pl.BlockSpec((2, _R2, _D), lambda r: (0, r, 0)),
            pl.BlockSpec((_R2, _D), lambda r: (r, 0)),
            pl.BlockSpec((_R2, _D), lambda r: (r, 0)),
            pl.BlockSpec((_R2, 1), lambda r: (r, 0)),
            pl.BlockSpec((_D, _D), lambda r: (0, 0)),
            pl.BlockSpec((_D, _O), lambda r: (0, 0)),
            pl.BlockSpec((1, _O), lambda r: (0, 0)),
        ],
        out_specs=[
            pl.BlockSpec((_R2, _O), lambda r: (r, 0)),
            pl.BlockSpec((_R2, _D), lambda r: (r, 0)),
            pl.BlockSpec((_R2, _O), lambda r: (r, 0)),
            pl.BlockSpec((_R2, 1), lambda r: (r, 0)),
        ],
        out_shape=[
            jax.ShapeDtypeStruct((_N, _O), jnp.float32),
            jax.ShapeDtypeStruct((_N, _D), jnp.float32),
            jax.ShapeDtypeStruct((_N, _O), jnp.float32),
            jax.ShapeDtypeStruct((_N, 1), jnp.int32),
        ],
    )(acc3, xs1, x0, dinv, W2, Wc, bc2)


def kernel(x, edge_index, Wp, bp, W1, W2, Wc, bc):
    src = edge_index[0]
    dst = edge_index[1]
    fill = jnp.full((_EPAD - _E,), _N, jnp.int32)
    src_p = jnp.concatenate([src, fill])
    dst_p = jnp.concatenate([dst, fill])
    x_pad = jnp.zeros((_NPAD, _D), jnp.float32).at[:_N].set(x)

    degs = _sc_degree(dst_p).reshape(2, _NPAD, 1)
    x0, xs0, dinv = _dense0(x_pad, Wp, bp.reshape(1, _D), degs)
    acc1 = _sc_propagate(xs0, src_p, dst_p).reshape(2, _NPAD, _D)
    xs1 = _combine1(acc1, xs0, x0, dinv, W1)
    acc2 = _sc_propagate(xs1, src_p, dst_p).reshape(2, _NPAD, _D)
    logits, emb, soft, hard = _head(acc2, xs1, x0, dinv, W2, Wc,
                                    bc.reshape(1, _O))
    return (logits, emb, soft, jnp.squeeze(hard, -1))
